# baseline (speedup 1.0000x reference)
import jax
import jax.numpy as jnp
from jax import lax
from jax.experimental import pallas as pl
from jax.experimental.pallas import tpu as pltpu
from jax.experimental.pallas import tpu_sc as plsc

N_SLOTS = 1048576
D_STATE = 8
B = 1048576
MOMENTUM = 0.9

NC = 2
NS = 16
HALF = N_SLOTS // NC
TRASH = 4096
ACC = HALF + TRASH
PER_TILE = B // NS
W = 2048
NWIN = PER_TILE // W
NPASS = 5
ZWORDS = 2 * ACC // NS
WBROWS = HALF // NS
WBCH = 1024


def _addr_body(b1_ref, b2_ref, b3_ref, o_ref):
    b1 = b1_ref[...]
    b2 = b2_ref[...]
    b3 = b3_ref[...]
    o_ref[...] = ((b1 & 15) << 16) | (b2 << 8) | b3


def _compute_addr(b1, b2, b3):
    nb = b1.shape[0]
    blk = 512
    grid = nb // blk
    return pl.pallas_call(
        _addr_body,
        grid=(grid,),
        in_specs=[pl.BlockSpec((blk, 128), lambda i: (i, i * 0))] * 3,
        out_specs=pl.BlockSpec((blk, 128), lambda i: (i, i * 0)),
        out_shape=jax.ShapeDtypeStruct((nb, 128), jnp.int32),
    )(b1, b2, b3)


def _sc_body(addr_hbm, states_hbm, sums_hbm, hits_hbm,
             ab0, ab1, ix0, ix1, rb0, rb1, pb0, pb1, wbin, cb0, cb1, zbuf,
             acc, asem0, asem1, rsem0, rsem1, psem0, psem1):
    core = lax.axis_index("c")
    sub = lax.axis_index("s")
    sbase = sub * PER_TILE
    slot_base = core * HALF
    iota = lax.iota(jnp.int32, 16)
    abs_ = (ab0, ab1)
    ixs = (ix0, ix1)
    rbs = (rb0, rb1)
    pbs = (pb0, pb1)
    cbs = (cb0, cb1)
    asems = (asem0, asem1)
    rsems = (rsem0, rsem1)
    psems = (psem0, psem1)

    half_iota = jnp.right_shift(iota, 1)
    parity = jnp.bitwise_and(iota, 1)

    def _zinit(j, _):
        idx = pl.ds(pl.multiple_of(j * 16, 16), 16)
        zbuf[idx] = jnp.full((16,), 0.0, jnp.float32)
        return _
    lax.fori_loop(jnp.int32(0), jnp.int32(zbuf.shape[0] // 16), _zinit, None)

    def _ard(w, b):
        return pltpu.make_async_copy(
            addr_hbm.at[pl.ds(sbase + w * W, W)], abs_[b], asems[b])

    def _rowrd(w, b):
        return pltpu.make_async_copy(
            states_hbm.at[pl.ds(8 * (sbase + w * W), 8 * W)], rbs[b],
            rsems[b])

    def _zero_acc():
        zoff = sub * ZWORDS
        zch = zbuf.shape[0]

        def _z(c, _):
            pltpu.sync_copy(zbuf.at[pl.ds(0, zch)],
                            acc.at[pl.ds(zoff + c * zch, zch)])
            return _
        lax.fori_loop(jnp.int32(0), jnp.int32(ZWORDS // zch), _z, None)
        rem = ZWORDS % zch
        if rem:
            pltpu.sync_copy(zbuf.at[pl.ds(0, rem)],
                            acc.at[pl.ds(zoff + (ZWORDS // zch) * zch, rem)])

    def _index_win(w, b):
        ab = abs_[b]
        ix = ixs[b]

        def _vec(j, _):
            idx = pl.ds(pl.multiple_of(j * 16, 16), 16)
            a = ab[idx]
            li = a - slot_base
            ok = li.astype(jnp.uint32) < jnp.uint32(HALF)
            trash = (HALF + ((j * 16) & (TRASH - 1))) + iota
            d = 2 * jnp.where(ok, li, trash)
            pos = j * 32 + 2 * iota
            plsc.store_scatter(ix, [pos], d)
            plsc.store_scatter(ix, [pos + 1], d + 1)
            return _
        lax.fori_loop(jnp.int32(0), jnp.int32(W // 16), _vec, None)

    def _extract_pair(p, b):
        rb = rbs[b]
        pb = pbs[b]
        dimv = 2 * p + parity

        def _v(v, _):
            fidx = (v * 8 + half_iota) * 8 + dimv
            x = plsc.load_gather(rb, [fidx])
            pb[pl.ds(pl.multiple_of(v * 16, 16), 16)] = x
            return _
        lax.fori_loop(jnp.int32(0), jnp.int32(W * 2 // 16), _v, None)

    def _writeback(p, rows):
        wb = sub * WBROWS

        def _chunk(part, _):
            o = wb + part * WBCH
            pltpu.sync_copy(acc.at[pl.ds(2 * o, 2 * WBCH)], wbin)
            for j in range(rows):
                def _x(v, __):
                    fidx = (v * 16 + iota) * 2 + j
                    x = plsc.load_gather(wbin, [fidx])
                    cbs[j][pl.ds(pl.multiple_of(v * 16, 16), 16)] = x
                    return __
                lax.fori_loop(jnp.int32(0), jnp.int32(WBCH // 16), _x, None)
                if p < 4:
                    dst = sums_hbm.at[jnp.int32(2 * p + j),
                                      pl.ds(slot_base + o, WBCH)]
                else:
                    dst = hits_hbm.at[jnp.int32(0),
                                      pl.ds(slot_base + o, WBCH)]
                pltpu.sync_copy(cbs[j], dst)
            return _
        lax.fori_loop(jnp.int32(0), jnp.int32(WBROWS // WBCH), _chunk, None)

    for p in range(NPASS):
        plsc.subcore_barrier()
        _zero_acc()
        if p == 4:
            ones_pair = jnp.where(parity == 0, jnp.float32(1.0),
                                  jnp.float32(0.0))

            def _of(v, _):
                pb0[pl.ds(pl.multiple_of(v * 16, 16), 16)] = ones_pair
                return _
            lax.fori_loop(jnp.int32(0), jnp.int32(W * 2 // 16), _of, None)
        plsc.subcore_barrier()
        _ard(jnp.int32(0), 0).start()
        _ard(jnp.int32(1), 1).start()
        if p < 4:
            _rowrd(jnp.int32(0), 0).start()
            _rowrd(jnp.int32(1), 1).start()

        def _pairwin(wi, _):
            scs = []
            for b in range(2):
                w = wi * 2 + b
                _ard(w, b).wait()
                if p < 4:
                    _rowrd(w, b).wait()
                _index_win(w, b)
                if p < 4:
                    _extract_pair(p, b)
                src = pbs[b] if p < 4 else pb0
                scs.append(pltpu.async_copy(
                    src, acc.at[ixs[b]], psems[b], add=True))

                @pl.when(w + 2 < NWIN)
                def _starts():
                    _ard(w + 2, b).start()
                    if p < 4:
                        _rowrd(w + 2, b).start()
            scs[0].wait()
            scs[1].wait()
            return _
        lax.fori_loop(jnp.int32(0), jnp.int32(NWIN // 2), _pairwin, None)
        plsc.subcore_barrier()
        _writeback(p, 2 if p < 4 else 1)


def _sc_scatter(addr, states_flat):
    mesh = plsc.VectorSubcoreMesh(core_axis_name="c", subcore_axis_name="s")
    kern = pl.kernel(
        _sc_body,
        out_type=[
            jax.ShapeDtypeStruct((D_STATE, N_SLOTS), jnp.float32),
            jax.ShapeDtypeStruct((1, N_SLOTS), jnp.float32),
        ],
        mesh=mesh,
        compiler_params=pltpu.CompilerParams(needs_layout_passes=False),
        scratch_types=[
            pltpu.VMEM((W,), jnp.int32),
            pltpu.VMEM((W,), jnp.int32),
            pltpu.VMEM((W * 2,), jnp.int32),
            pltpu.VMEM((W * 2,), jnp.int32),
            pltpu.VMEM((W * D_STATE,), jnp.float32),
            pltpu.VMEM((W * D_STATE,), jnp.float32),
            pltpu.VMEM((W * 2,), jnp.float32),
            pltpu.VMEM((W * 2,), jnp.float32),
            pltpu.VMEM((2 * WBCH,), jnp.float32),
            pltpu.VMEM((WBCH,), jnp.float32),
            pltpu.VMEM((WBCH,), jnp.float32),
            pltpu.VMEM((2048,), jnp.float32),
            pltpu.VMEM_SHARED((2 * ACC,), jnp.float32),
            pltpu.SemaphoreType.DMA,
            pltpu.SemaphoreType.DMA,
            pltpu.SemaphoreType.DMA,
            pltpu.SemaphoreType.DMA,
            pltpu.SemaphoreType.DMA,
            pltpu.SemaphoreType.DMA,
        ],
    )
    return kern(addr, states_flat)


def _merge_body(s_ref, h_ref, c_ref, b_ref, nb_ref, nc_ref):
    f32 = jnp.float32
    sums = s_ref[...]
    hits = h_ref[...]
    hit = hits > f32(0.0)
    mean = sums / jnp.maximum(hits, f32(1.0))
    cnt = c_ref[...]
    alpha = jnp.where(cnt == 0, f32(0.0), f32(MOMENTUM))
    av = jnp.where(hit, alpha, f32(1.0))
    wv = jnp.where(hit, f32(1.0) - alpha, f32(0.0))
    nb_ref[...] = b_ref[...] * av + mean * wv
    nc_ref[...] = cnt + hits.astype(jnp.int32)


def _merge(sumsT, hits, counts32, bankT):
    bs = 16384
    grid = N_SLOTS // bs
    return pl.pallas_call(
        _merge_body,
        grid=(grid,),
        in_specs=[
            pl.BlockSpec((D_STATE, bs), lambda i: (i * 0, i)),
            pl.BlockSpec((1, bs), lambda i: (i * 0, i)),
            pl.BlockSpec((1, bs), lambda i: (i * 0, i)),
            pl.BlockSpec((D_STATE, bs), lambda i: (i * 0, i)),
        ],
        out_specs=[
            pl.BlockSpec((D_STATE, bs), lambda i: (i * 0, i)),
            pl.BlockSpec((1, bs), lambda i: (i * 0, i)),
        ],
        out_shape=[
            jax.ShapeDtypeStruct((D_STATE, N_SLOTS), jnp.float32),
            jax.ShapeDtypeStruct((1, N_SLOTS), jnp.int32),
        ],
    )(sumsT, hits, counts32, bankT)


def kernel(byte_window, states, bank, counts):
    bw32 = byte_window.astype(jnp.int32)
    nb = B // 128
    b1 = bw32[:, 1].reshape(nb, 128)
    b2 = bw32[:, 2].reshape(nb, 128)
    b3 = bw32[:, 3].reshape(nb, 128)
    addr = _compute_addr(b1, b2, b3).reshape(B)

    states_flat = states.astype(jnp.float32).reshape(B * D_STATE)
    sumsT, hits = _sc_scatter(addr, states_flat)

    counts32 = counts.astype(jnp.int32).reshape(1, N_SLOTS)
    bankT = bank.T
    nbT, ncl = _merge(sumsT, hits, counts32, bankT)
    new_bank = nbT.T
    new_counts = ncl.reshape(N_SLOTS).astype(jnp.int64)
    return new_bank, new_counts


# speedup vs baseline: 2.2742x; 2.2742x over previous
import jax
import jax.numpy as jnp
from jax import lax
from jax.experimental import pallas as pl
from jax.experimental.pallas import tpu as pltpu
from jax.experimental.pallas import tpu_sc as plsc

N_SLOTS = 1048576
D_STATE = 8
B = 1048576
MOMENTUM = 0.9

NC = 2
NS = 16
HALF = N_SLOTS // NC
TRASH = 16384
ACC = HALF + TRASH
PER_TILE = B // NS
W = 8192
NWIN = PER_TILE // W
ZSPAN = ACC // NS
ZCH = 4096
WBSPAN = HALF // NS


def _addr_body(b1_ref, b2_ref, b3_ref, o_ref):
    b1 = b1_ref[...]
    b2 = b2_ref[...]
    b3 = b3_ref[...]
    o_ref[...] = ((b1 & 15) << 16) | (b2 << 8) | b3


def _compute_addr(b1, b2, b3):
    nb = b1.shape[0]
    blk = 512
    grid = nb // blk
    return pl.pallas_call(
        _addr_body,
        grid=(grid,),
        in_specs=[pl.BlockSpec((blk, 128), lambda i: (i, i * 0))] * 3,
        out_specs=pl.BlockSpec((blk, 128), lambda i: (i, i * 0)),
        out_shape=jax.ShapeDtypeStruct((nb, 128), jnp.int32),
    )(b1, b2, b3)


def _sc_body(addr_hbm, statesT_hbm, sums_hbm, hits_hbm,
             li_buf, buf0, buf1, wb_buf, zbuf, acc, sem0, sem1):
    core = lax.axis_index("c")
    sub = lax.axis_index("s")
    sbase = sub * PER_TILE
    slot_base = core * HALF
    iota = lax.iota(jnp.int32, 16)
    bufs = (buf0, buf1)
    sems = (sem0, sem1)

    def _init(j, _):
        idx = pl.ds(pl.multiple_of(j * 16, 16), 16)
        zbuf[idx] = jnp.full((16,), 0.0, jnp.float32)
        return _
    lax.fori_loop(jnp.int32(0), jnp.int32(ZCH // 16), _init, None)

    def _addr_cp(w):
        return pltpu.make_async_copy(
            addr_hbm.at[pl.ds(sbase + w * W, W)],
            li_buf.at[pl.ds(w * W, W)], sems[w % 2])

    _addr_cp(0).start()
    for w in range(NWIN):
        _addr_cp(w).wait()
        if w + 1 < NWIN:
            _addr_cp(w + 1).start()

        def _vec(j, _):
            idx = pl.ds(pl.multiple_of(w * W + j * 16, 16), 16)
            a = li_buf[idx]
            li = a - slot_base
            ok = li.astype(jnp.uint32) < jnp.uint32(HALF)
            trash = (HALF + ((j * 16) & (TRASH - 1))) + iota
            li_buf[idx] = jnp.where(ok, li, trash)
            return _
        lax.fori_loop(jnp.int32(0), jnp.int32(W // 16), _vec, None)

    def _st_cp(k):
        col, w = divmod(k, NWIN)
        return pltpu.make_async_copy(
            statesT_hbm.at[jnp.int32(col), pl.ds(sbase + w * W, W)],
            bufs[k % 2], sems[k % 2])

    def _zero_acc():
        zoff = sub * ZSPAN
        off = 0
        for sz in (ZCH,) * (ZSPAN // ZCH) + (ZSPAN % ZCH,):
            pltpu.sync_copy(zbuf.at[pl.ds(0, sz)],
                            acc.at[pl.ds(zoff + off, sz)])
            off += sz

    def _writeback(out_ref, row):
        wb = sub * WBSPAN
        for part in range(WBSPAN // W):
            o = wb + part * W
            pltpu.sync_copy(acc.at[pl.ds(o, W)], wb_buf)
            pltpu.sync_copy(
                wb_buf, out_ref.at[jnp.int32(row), pl.ds(slot_base + o, W)])

    NK = D_STATE * NWIN
    _st_cp(0).start()
    _st_cp(1).start()
    for col in range(D_STATE):
        plsc.subcore_barrier()
        _zero_acc()
        plsc.subcore_barrier()
        for w in range(NWIN):
            k = col * NWIN + w
            _st_cp(k).wait()
            pltpu.sync_copy(bufs[k % 2],
                            acc.at[li_buf.at[pl.ds(w * W, W)]], add=True)
            if k + 2 < NK:
                _st_cp(k + 2).start()
        plsc.subcore_barrier()
        _writeback(sums_hbm, col)

    def _ones(j, _):
        idx = pl.ds(pl.multiple_of(j * 16, 16), 16)
        buf0[idx] = jnp.full((16,), 1.0, jnp.float32)
        return _
    lax.fori_loop(jnp.int32(0), jnp.int32(W // 16), _ones, None)
    plsc.subcore_barrier()
    _zero_acc()
    plsc.subcore_barrier()
    for w in range(NWIN):
        pltpu.sync_copy(buf0, acc.at[li_buf.at[pl.ds(w * W, W)]], add=True)
    plsc.subcore_barrier()
    _writeback(hits_hbm, 0)


def _sc_scatter(addr, statesT):
    mesh = plsc.VectorSubcoreMesh(core_axis_name="c", subcore_axis_name="s")
    kern = pl.kernel(
        _sc_body,
        out_type=[
            jax.ShapeDtypeStruct((D_STATE, N_SLOTS), jnp.float32),
            jax.ShapeDtypeStruct((1, N_SLOTS), jnp.float32),
        ],
        mesh=mesh,
        scratch_types=[
            pltpu.VMEM((PER_TILE,), jnp.int32),
            pltpu.VMEM((W,), jnp.float32),
            pltpu.VMEM((W,), jnp.float32),
            pltpu.VMEM((W,), jnp.float32),
            pltpu.VMEM((ZCH,), jnp.float32),
            pltpu.VMEM_SHARED((ACC,), jnp.float32),
            pltpu.SemaphoreType.DMA,
            pltpu.SemaphoreType.DMA,
        ],
    )
    return kern(addr, statesT)


def _merge_body(s_ref, h_ref, c_ref, b_ref, nb_ref, nc_ref):
    f32 = jnp.float32
    sums = s_ref[...]
    hits = h_ref[...]
    hit = hits > f32(0.0)
    mean = sums / jnp.maximum(hits, f32(1.0))
    cnt = c_ref[...]
    alpha = jnp.where(cnt == 0, f32(0.0), f32(MOMENTUM))
    av = jnp.where(hit, alpha, f32(1.0))
    wv = jnp.where(hit, f32(1.0) - alpha, f32(0.0))
    nb_ref[...] = b_ref[...] * av + mean * wv
    nc_ref[...] = cnt + hits.astype(jnp.int32)


def _merge(sumsT, hits, counts32, bankT):
    bs = 16384
    grid = N_SLOTS // bs
    return pl.pallas_call(
        _merge_body,
        grid=(grid,),
        in_specs=[
            pl.BlockSpec((D_STATE, bs), lambda i: (i * 0, i)),
            pl.BlockSpec((1, bs), lambda i: (i * 0, i)),
            pl.BlockSpec((1, bs), lambda i: (i * 0, i)),
            pl.BlockSpec((D_STATE, bs), lambda i: (i * 0, i)),
        ],
        out_specs=[
            pl.BlockSpec((D_STATE, bs), lambda i: (i * 0, i)),
            pl.BlockSpec((1, bs), lambda i: (i * 0, i)),
        ],
        out_shape=[
            jax.ShapeDtypeStruct((D_STATE, N_SLOTS), jnp.float32),
            jax.ShapeDtypeStruct((1, N_SLOTS), jnp.int32),
        ],
    )(sumsT, hits, counts32, bankT)


def kernel(byte_window, states, bank, counts):
    bw32 = byte_window.astype(jnp.int32)
    nb = B // 128
    b1 = bw32[:, 1].reshape(nb, 128)
    b2 = bw32[:, 2].reshape(nb, 128)
    b3 = bw32[:, 3].reshape(nb, 128)
    addr = _compute_addr(b1, b2, b3).reshape(B)

    statesT = states.astype(jnp.float32).T
    sumsT, hits = _sc_scatter(addr, statesT)

    counts32 = counts.astype(jnp.int32).reshape(1, N_SLOTS)
    bankT = bank.T
    nbT, ncl = _merge(sumsT, hits, counts32, bankT)
    new_bank = nbT.T
    new_counts = ncl.reshape(N_SLOTS).astype(jnp.int64)
    return new_bank, new_counts
